# baseline (speedup 1.0000x reference)
import functools

import jax
import jax.numpy as jnp
from jax import lax
from jax.experimental import pallas as pl
from jax.experimental.pallas import tpu as pltpu
from jax.experimental.pallas import tpu_sc as plsc

_N = 10000
_E = 320000
_D = 128

_NC = 2
_NS = 16
_NW = _NC * _NS
_EPT = _E // _NW
_CHUNK = 128
_NCHUNKS = 80
_EPTP = _NCHUNKS * _CHUNK
_NG = _NCHUNKS // 4
_NPAD = 10016
_RPT = 624
_REM = _N - _NS * _RPT
_CNTW = _D


_MESH = plsc.VectorSubcoreMesh(
    core_axis_name="c", subcore_axis_name="s",
    num_cores=_NC, num_subcores=_NS)


def _sc_agg_body(h_hbm, src_flat, edges_hbm, z128_hbm, agg_out,
                 dst_v, sa, sb, rows_a, rows_b, acc_sh,
                 gsem_a, gsem_b, isem):
  cid = lax.axis_index("c")
  sid = lax.axis_index("s")
  wid = sid * _NC + cid

  pltpu.sync_copy(edges_hbm.at[1, wid], dst_v)

  r0 = pl.multiple_of(sid * _RPT, 8)
  pltpu.sync_copy(z128_hbm, acc_sh.at[pl.ds(r0, _RPT)])

  @pl.when(sid == 0)
  def _zero_rem():
    pltpu.sync_copy(z128_hbm.at[pl.ds(0, _REM)],
                    acc_sh.at[pl.ds(_NS * _RPT, _REM)])

  plsc.subcore_barrier()

  def idx(c, buf):
    off = pl.multiple_of(wid * _EPTP + c * _CHUNK, _CHUNK)
    pltpu.async_copy(src_flat.at[pl.ds(off, _CHUNK)], buf, isem).wait()

  def gwait(buf, rows, sem):
    pltpu.make_async_copy(h_hbm.at[buf], rows, sem).wait()

  idx(0, sa)
  pltpu.async_copy(h_hbm.at[sa], rows_a, gsem_a)

  def step(i, carry):
    c0 = i * 2
    idx(c0 + 1, sb)
    pltpu.async_copy(h_hbm.at[sb], rows_b, gsem_b)
    gwait(sa, rows_a, gsem_a)
    pltpu.sync_copy(rows_a, acc_sh.at[dst_v.at[c0]], add=True)
    idx(jnp.minimum(c0 + 2, _NCHUNKS - 1), sa)
    pltpu.async_copy(h_hbm.at[sa], rows_a, gsem_a)
    gwait(sb, rows_b, gsem_b)
    pltpu.sync_copy(rows_b, acc_sh.at[dst_v.at[c0 + 1]], add=True)
    return carry

  lax.fori_loop(0, _NCHUNKS // 2 - 1, step, 0)
  gwait(sa, rows_a, gsem_a)
  pltpu.sync_copy(rows_a, acc_sh.at[dst_v.at[_NCHUNKS - 2]], add=True)
  idx(_NCHUNKS - 1, sb)
  pltpu.async_copy(h_hbm.at[sb], rows_b, gsem_b).wait()
  pltpu.sync_copy(rows_b, acc_sh.at[dst_v.at[_NCHUNKS - 1]], add=True)
  plsc.subcore_barrier()

  rows = pl.ds(r0, _RPT)
  pltpu.sync_copy(acc_sh.at[rows], agg_out.at[cid, rows])

  @pl.when(sid == 0)
  def _copy_rem():
    rem = pl.ds(_NS * _RPT, _REM)
    pltpu.sync_copy(acc_sh.at[rem], agg_out.at[cid, rem])


_sc_agg = pl.kernel(
    _sc_agg_body,
    out_type=jax.ShapeDtypeStruct((_NC, _N, _D), jnp.float32),
    mesh=_MESH,
    scratch_types=[
        pltpu.VMEM((_NCHUNKS, _CHUNK), jnp.int32),
        pltpu.VMEM((_CHUNK,), jnp.int32),
        pltpu.VMEM((_CHUNK,), jnp.int32),
        pltpu.VMEM((_CHUNK, _D), jnp.float32),
        pltpu.VMEM((_CHUNK, _D), jnp.float32),
        pltpu.VMEM_SHARED((_NPAD, _D), jnp.float32),
        pltpu.SemaphoreType.DMA,
        pltpu.SemaphoreType.DMA,
        pltpu.SemaphoreType.DMA,
    ],
    name="sc_sage_agg")


def _sc_counts_body(edges_hbm, z16_hbm, ones_hbm, cnt_out,
                    dst_v, ones_v, cnt_sh, sem):
  cid = lax.axis_index("c")
  sid = lax.axis_index("s")
  wid = sid * _NC + cid

  pltpu.sync_copy(edges_hbm.at[1, wid], dst_v)
  pltpu.sync_copy(ones_hbm, ones_v)

  r0 = pl.multiple_of(sid * _RPT, 8)
  pltpu.sync_copy(z16_hbm, cnt_sh.at[pl.ds(r0, _RPT)])

  @pl.when(sid == 0)
  def _zero_rem():
    pltpu.sync_copy(z16_hbm.at[pl.ds(0, _REM)],
                    cnt_sh.at[pl.ds(_NS * _RPT, _REM)])

  plsc.subcore_barrier()

  def step(c, carry):
    pltpu.async_copy(ones_v, cnt_sh.at[dst_v.at[c]], sem, add=True)
    return carry

  def drain(c, carry):
    pltpu.make_async_copy(ones_v, cnt_sh.at[dst_v.at[0]], sem).wait()
    return carry

  lax.fori_loop(0, _NCHUNKS, step, 0)
  lax.fori_loop(0, _NCHUNKS, drain, 0)
  plsc.subcore_barrier()

  rows = pl.ds(r0, _RPT)
  pltpu.sync_copy(cnt_sh.at[rows], cnt_out.at[cid, rows])

  @pl.when(sid == 0)
  def _copy_rem():
    rem = pl.ds(_NS * _RPT, _REM)
    pltpu.sync_copy(cnt_sh.at[rem], cnt_out.at[cid, rem])


_sc_counts = pl.kernel(
    _sc_counts_body,
    out_type=jax.ShapeDtypeStruct((_NC, _N, _CNTW), jnp.float32),
    mesh=_MESH,
    scratch_types=[
        pltpu.VMEM((_NCHUNKS, _CHUNK), jnp.int32),
        pltpu.VMEM((_CHUNK, _CNTW), jnp.float32),
        pltpu.VMEM_SHARED((_NPAD, _CNTW), jnp.float32),
        pltpu.SemaphoreType.DMA,
    ],
    name="sc_sage_counts")


_BN = 1000
_GRID = _N // _BN


def _dotT(x, w):
  return lax.dot_general(x, w, (((1,), (1,)), ((), ())),
                         preferred_element_type=jnp.float32)


def _tc_pre_body(x_ref, wl_ref, wr_ref, b_ref, h_ref, r_ref):
  x = x_ref[...]
  h_ref[...] = _dotT(x, wl_ref[...])
  r_ref[...] = _dotT(x, wr_ref[...]) + b_ref[...]


_tc_pre = pl.pallas_call(
    _tc_pre_body,
    grid=(_GRID,),
    in_specs=[
        pl.BlockSpec((_BN, _D), lambda i: (i, 0)),
        pl.BlockSpec((_D, _D), lambda i: (0, 0)),
        pl.BlockSpec((_D, _D), lambda i: (0, 0)),
        pl.BlockSpec((1, _D), lambda i: (0, 0)),
    ],
    out_specs=[pl.BlockSpec((_BN, _D), lambda i: (i, 0))] * 2,
    out_shape=[jax.ShapeDtypeStruct((_N, _D), jnp.float32)] * 2,
)


def _tc_mid_body(agg_ref, cnt_ref, r1_ref, wl_ref, wr_ref, b_ref,
                 h_ref, r_ref):
  agg = agg_ref[0] + agg_ref[1]
  cnt = cnt_ref[0, :, 0:1] + cnt_ref[1, :, 0:1]
  x1 = jnp.maximum(agg / jnp.maximum(cnt, 1.0) + r1_ref[...], 0.0)
  h_ref[...] = _dotT(x1, wl_ref[...])
  r_ref[...] = _dotT(x1, wr_ref[...]) + b_ref[...]


_tc_mid = pl.pallas_call(
    _tc_mid_body,
    grid=(_GRID,),
    in_specs=[
        pl.BlockSpec((_NC, _BN, _D), lambda i: (0, i, 0)),
        pl.BlockSpec((_NC, _BN, _CNTW), lambda i: (0, i, 0)),
        pl.BlockSpec((_BN, _D), lambda i: (i, 0)),
        pl.BlockSpec((_D, _D), lambda i: (0, 0)),
        pl.BlockSpec((_D, _D), lambda i: (0, 0)),
        pl.BlockSpec((1, _D), lambda i: (0, 0)),
    ],
    out_specs=[pl.BlockSpec((_BN, _D), lambda i: (i, 0))] * 2,
    out_shape=[jax.ShapeDtypeStruct((_N, _D), jnp.float32)] * 2,
)


def _tc_post_body(agg_ref, cnt_ref, r2_ref, out_ref):
  agg = agg_ref[0] + agg_ref[1]
  cnt = cnt_ref[0, :, 0:1] + cnt_ref[1, :, 0:1]
  out_ref[...] = agg / jnp.maximum(cnt, 1.0) + r2_ref[...]


_tc_post = pl.pallas_call(
    _tc_post_body,
    grid=(_GRID,),
    in_specs=[
        pl.BlockSpec((_NC, _BN, _D), lambda i: (0, i, 0)),
        pl.BlockSpec((_NC, _BN, _CNTW), lambda i: (0, i, 0)),
        pl.BlockSpec((_BN, _D), lambda i: (i, 0)),
    ],
    out_specs=pl.BlockSpec((_BN, _D), lambda i: (i, 0)),
    out_shape=jax.ShapeDtypeStruct((_N, _D), jnp.float32),
)


@jax.jit
def kernel(node_features, edge_indices, W_l1, b_l1, W_r1, W_l2, b_l2, W_r2):
  x = node_features
  e3 = edge_indices.reshape(2, _NW, _EPT)
  src_p = jnp.pad(e3[0], ((0, 0), (0, _EPTP - _EPT)))
  dst_p = jnp.pad(e3[1], ((0, 0), (0, _EPTP - _EPT)), constant_values=_N)
  edges = jnp.stack([src_p, dst_p]).reshape(2, _NW, _NCHUNKS, _CHUNK)
  src_flat = src_p.reshape(_NW * _EPTP)
  z128 = jnp.zeros((_RPT, _D), jnp.float32)
  ones = jnp.ones((_CHUNK, _CNTW), jnp.float32)
  cnt = _sc_counts(edges, z128, ones)
  h1, r1 = _tc_pre(x, W_l1, W_r1, b_l1.reshape(1, _D))
  agg1 = _sc_agg(h1, src_flat, edges, z128)
  h2, r2 = _tc_mid(agg1, cnt, r1, W_l2, W_r2, b_l2.reshape(1, _D))
  agg2 = _sc_agg(h2, src_flat, edges, z128)
  return _tc_post(agg2, cnt, r2)


# speedup vs baseline: 1.0018x; 1.0018x over previous
import functools

import jax
import jax.numpy as jnp
from jax import lax
from jax.experimental import pallas as pl
from jax.experimental.pallas import tpu as pltpu
from jax.experimental.pallas import tpu_sc as plsc

_N = 10000
_E = 320000
_D = 128

_NC = 2
_NS = 16
_NW = _NC * _NS
_EPT = _E // _NW
_CHUNK = 128
_NCHUNKS = 80
_EPTP = _NCHUNKS * _CHUNK
_NG = _NCHUNKS // 4
_NPAD = 10016
_RPT = 624
_REM = _N - _NS * _RPT
_CNTW = _D


_MESH = plsc.VectorSubcoreMesh(
    core_axis_name="c", subcore_axis_name="s",
    num_cores=_NC, num_subcores=_NS)


def _sc_agg_body(h_hbm, src_flat, edges_hbm, z128_hbm, agg_out,
                 dst_v, sa, sb, rows_a, rows_b, acc_sh,
                 gsem_a, gsem_b, isem):
  cid = lax.axis_index("c")
  sid = lax.axis_index("s")
  wid = sid * _NC + cid

  pltpu.sync_copy(edges_hbm.at[1, wid], dst_v)

  r0 = pl.multiple_of(sid * _RPT, 8)
  pltpu.sync_copy(z128_hbm, acc_sh.at[pl.ds(r0, _RPT)])

  @pl.when(sid == 0)
  def _zero_rem():
    pltpu.sync_copy(z128_hbm.at[pl.ds(0, _REM)],
                    acc_sh.at[pl.ds(_NS * _RPT, _REM)])

  plsc.subcore_barrier()

  def idx(c, buf):
    off = pl.multiple_of(wid * _EPTP + c * _CHUNK, _CHUNK)
    pltpu.async_copy(src_flat.at[pl.ds(off, _CHUNK)], buf, isem).wait()

  def gwait(buf, rows, sem):
    pltpu.make_async_copy(h_hbm.at[buf], rows, sem).wait()

  idx(0, sa)
  pltpu.async_copy(h_hbm.at[sa], rows_a, gsem_a)

  def step(i, carry):
    c0 = i * 2
    idx(c0 + 1, sb)
    pltpu.async_copy(h_hbm.at[sb], rows_b, gsem_b)
    gwait(sa, rows_a, gsem_a)
    pltpu.sync_copy(rows_a, acc_sh.at[dst_v.at[c0]], add=True)
    idx(jnp.minimum(c0 + 2, _NCHUNKS - 1), sa)
    pltpu.async_copy(h_hbm.at[sa], rows_a, gsem_a)
    gwait(sb, rows_b, gsem_b)
    pltpu.sync_copy(rows_b, acc_sh.at[dst_v.at[c0 + 1]], add=True)
    return carry

  lax.fori_loop(0, _NCHUNKS // 2 - 1, step, 0)
  gwait(sa, rows_a, gsem_a)
  pltpu.sync_copy(rows_a, acc_sh.at[dst_v.at[_NCHUNKS - 2]], add=True)
  idx(_NCHUNKS - 1, sb)
  pltpu.async_copy(h_hbm.at[sb], rows_b, gsem_b).wait()
  pltpu.sync_copy(rows_b, acc_sh.at[dst_v.at[_NCHUNKS - 1]], add=True)
  plsc.subcore_barrier()

  rows = pl.ds(r0, _RPT)
  pltpu.sync_copy(acc_sh.at[rows], agg_out.at[cid, rows])

  @pl.when(sid == 0)
  def _copy_rem():
    rem = pl.ds(_NS * _RPT, _REM)
    pltpu.sync_copy(acc_sh.at[rem], agg_out.at[cid, rem])


_sc_agg = pl.kernel(
    _sc_agg_body,
    out_type=jax.ShapeDtypeStruct((_NC, _N, _D), jnp.float32),
    mesh=_MESH,
    scratch_types=[
        pltpu.VMEM((_NCHUNKS, _CHUNK), jnp.int32),
        pltpu.VMEM((_CHUNK,), jnp.int32),
        pltpu.VMEM((_CHUNK,), jnp.int32),
        pltpu.VMEM((_CHUNK, _D), jnp.float32),
        pltpu.VMEM((_CHUNK, _D), jnp.float32),
        pltpu.VMEM_SHARED((_NPAD, _D), jnp.float32),
        pltpu.SemaphoreType.DMA,
        pltpu.SemaphoreType.DMA,
        pltpu.SemaphoreType.DMA,
    ],
    name="sc_sage_agg")


def _sc_counts_body(edges_hbm, z16_hbm, ones_hbm, cnt_out,
                    dst_v, ones_v, cnt_sh, sem):
  cid = lax.axis_index("c")
  sid = lax.axis_index("s")
  wid = sid * _NC + cid

  pltpu.sync_copy(edges_hbm.at[1, wid], dst_v)
  pltpu.sync_copy(ones_hbm, ones_v)

  r0 = pl.multiple_of(sid * _RPT, 8)
  pltpu.sync_copy(z16_hbm, cnt_sh.at[pl.ds(r0, _RPT)])

  @pl.when(sid == 0)
  def _zero_rem():
    pltpu.sync_copy(z16_hbm.at[pl.ds(0, _REM)],
                    cnt_sh.at[pl.ds(_NS * _RPT, _REM)])

  plsc.subcore_barrier()

  def step(c, carry):
    pltpu.async_copy(ones_v, cnt_sh.at[dst_v.at[c]], sem, add=True)
    return carry

  def drain(c, carry):
    pltpu.make_async_copy(ones_v, cnt_sh.at[dst_v.at[0]], sem).wait()
    return carry

  lax.fori_loop(0, _NCHUNKS, step, 0)
  lax.fori_loop(0, _NCHUNKS, drain, 0)
  plsc.subcore_barrier()

  rows = pl.ds(r0, _RPT)
  pltpu.sync_copy(cnt_sh.at[rows], cnt_out.at[cid, rows])

  @pl.when(sid == 0)
  def _copy_rem():
    rem = pl.ds(_NS * _RPT, _REM)
    pltpu.sync_copy(cnt_sh.at[rem], cnt_out.at[cid, rem])


_sc_counts = pl.kernel(
    _sc_counts_body,
    out_type=jax.ShapeDtypeStruct((_NC, _N, _CNTW), jnp.float32),
    mesh=_MESH,
    scratch_types=[
        pltpu.VMEM((_NCHUNKS, _CHUNK), jnp.int32),
        pltpu.VMEM((_CHUNK, _CNTW), jnp.float32),
        pltpu.VMEM_SHARED((_NPAD, _CNTW), jnp.float32),
        pltpu.SemaphoreType.DMA,
    ],
    name="sc_sage_counts")


_BN = 1000
_GRID = _N // _BN


def _dotT(x, w):
  return lax.dot_general(x, w, (((1,), (1,)), ((), ())),
                         preferred_element_type=jnp.float32)


def _tc_pre_body(x_ref, wl_ref, wr_ref, b_ref, h_ref, r_ref):
  x = x_ref[...]
  h_ref[...] = _dotT(x, wl_ref[...])
  r_ref[...] = _dotT(x, wr_ref[...]) + b_ref[...]


_tc_pre = pl.pallas_call(
    _tc_pre_body,
    grid=(_GRID,),
    in_specs=[
        pl.BlockSpec((_BN, _D), lambda i: (i, 0)),
        pl.BlockSpec((_D, _D), lambda i: (0, 0)),
        pl.BlockSpec((_D, _D), lambda i: (0, 0)),
        pl.BlockSpec((1, _D), lambda i: (0, 0)),
    ],
    out_specs=[pl.BlockSpec((_BN, _D), lambda i: (i, 0))] * 2,
    out_shape=[jax.ShapeDtypeStruct((_N, _D), jnp.float32)] * 2,
)


def _tc_mid_body(agg_ref, cnt_ref, r1_ref, wl_ref, wr_ref, b_ref,
                 h_ref, r_ref):
  agg = agg_ref[0] + agg_ref[1]
  cnt = cnt_ref[0, :, 0:1] + cnt_ref[1, :, 0:1]
  x1 = jnp.maximum(agg / jnp.maximum(cnt, 1.0) + r1_ref[...], 0.0)
  h_ref[...] = _dotT(x1, wl_ref[...])
  r_ref[...] = _dotT(x1, wr_ref[...]) + b_ref[...]


_tc_mid = pl.pallas_call(
    _tc_mid_body,
    grid=(_GRID,),
    in_specs=[
        pl.BlockSpec((_NC, _BN, _D), lambda i: (0, i, 0)),
        pl.BlockSpec((_NC, _BN, _CNTW), lambda i: (0, i, 0)),
        pl.BlockSpec((_BN, _D), lambda i: (i, 0)),
        pl.BlockSpec((_D, _D), lambda i: (0, 0)),
        pl.BlockSpec((_D, _D), lambda i: (0, 0)),
        pl.BlockSpec((1, _D), lambda i: (0, 0)),
    ],
    out_specs=[pl.BlockSpec((_BN, _D), lambda i: (i, 0))] * 2,
    out_shape=[jax.ShapeDtypeStruct((_N, _D), jnp.float32)] * 2,
)


def _tc_post_body(agg_ref, cnt_ref, r2_ref, out_ref):
  agg = agg_ref[0] + agg_ref[1]
  cnt = cnt_ref[0, :, 0:1] + cnt_ref[1, :, 0:1]
  out_ref[...] = agg / jnp.maximum(cnt, 1.0) + r2_ref[...]


_tc_post = pl.pallas_call(
    _tc_post_body,
    grid=(_GRID,),
    in_specs=[
        pl.BlockSpec((_NC, _BN, _D), lambda i: (0, i, 0)),
        pl.BlockSpec((_NC, _BN, _CNTW), lambda i: (0, i, 0)),
        pl.BlockSpec((_BN, _D), lambda i: (i, 0)),
    ],
    out_specs=pl.BlockSpec((_BN, _D), lambda i: (i, 0)),
    out_shape=jax.ShapeDtypeStruct((_N, _D), jnp.float32),
)


@jax.jit
def kernel(node_features, edge_indices, W_l1, b_l1, W_r1, W_l2, b_l2, W_r2):
  x = node_features
  e3 = edge_indices.reshape(2, _NW, _EPT)
  src_p = jnp.pad(e3[0], ((0, 0), (0, _EPTP - _EPT)))
  trash = _N + (jnp.arange(_NW, dtype=jnp.int32) // _NC)[:, None]
  dst_p = jnp.concatenate(
      [e3[1], jnp.broadcast_to(trash, (_NW, _EPTP - _EPT))], axis=1)
  edges = jnp.stack([src_p, dst_p]).reshape(2, _NW, _NCHUNKS, _CHUNK)
  src_flat = src_p.reshape(_NW * _EPTP)
  z128 = jnp.zeros((_RPT, _D), jnp.float32)
  ones = jnp.ones((_CHUNK, _CNTW), jnp.float32)
  cnt = _sc_counts(edges, z128, ones)
  h1, r1 = _tc_pre(x, W_l1, W_r1, b_l1.reshape(1, _D))
  agg1 = _sc_agg(h1, src_flat, edges, z128)
  h2, r2 = _tc_mid(agg1, cnt, r1, W_l2, W_r2, b_l2.reshape(1, _D))
  agg2 = _sc_agg(h2, src_flat, edges, z128)
  return _tc_post(agg2, cnt, r2)


# speedup vs baseline: 2.4048x; 2.4004x over previous
import functools

import jax
import jax.numpy as jnp
from jax import lax
from jax.experimental import pallas as pl
from jax.experimental.pallas import tpu as pltpu
from jax.experimental.pallas import tpu_sc as plsc

_N = 10000
_E = 320000
_D = 128

_NC = 2
_NS = 16
_NW = _NC * _NS
_EPT = _E // _NW
_CHUNK = 128
_NCHUNKS = 80
_EPTP = _NCHUNKS * _CHUNK
_NG = _NCHUNKS // 4
_NPAD = 10016
_RPT = 624
_REM = _N - _NS * _RPT
_CNTW = _D


_MESH = plsc.VectorSubcoreMesh(
    core_axis_name="c", subcore_axis_name="s",
    num_cores=_NC, num_subcores=_NS)


def _sc_agg_body(h_hbm, src_flat, edges_hbm, z128_hbm, agg_out,
                 dst_v, sa, sb, rows_a, rows_b, acc_sh,
                 gsem_a, gsem_b, isem):
  cid = lax.axis_index("c")
  sid = lax.axis_index("s")
  wid = sid * _NC + cid

  pltpu.sync_copy(edges_hbm.at[1, wid], dst_v)

  r0 = pl.multiple_of(sid * _RPT, 8)
  pltpu.sync_copy(z128_hbm, acc_sh.at[pl.ds(r0, _RPT)])

  @pl.when(sid == 0)
  def _zero_rem():
    pltpu.sync_copy(z128_hbm.at[pl.ds(0, _REM)],
                    acc_sh.at[pl.ds(_NS * _RPT, _REM)])

  plsc.subcore_barrier()

  def idx(c, buf):
    off = pl.multiple_of(wid * _EPTP + c * _CHUNK, _CHUNK)
    pltpu.async_copy(src_flat.at[pl.ds(off, _CHUNK)], buf, isem).wait()

  def gwait(buf, rows, sem):
    pltpu.make_async_copy(h_hbm.at[buf], rows, sem).wait()

  idx(0, sa)
  pltpu.async_copy(h_hbm.at[sa], rows_a, gsem_a)

  def step(i, carry):
    c0 = i * 2
    idx(c0 + 1, sb)
    pltpu.async_copy(h_hbm.at[sb], rows_b, gsem_b)
    gwait(sa, rows_a, gsem_a)
    pltpu.sync_copy(rows_a, acc_sh.at[dst_v.at[c0]], add=True)
    idx(jnp.minimum(c0 + 2, _NCHUNKS - 1), sa)
    pltpu.async_copy(h_hbm.at[sa], rows_a, gsem_a)
    gwait(sb, rows_b, gsem_b)
    pltpu.sync_copy(rows_b, acc_sh.at[dst_v.at[c0 + 1]], add=True)
    return carry

  lax.fori_loop(0, _NCHUNKS // 2 - 1, step, 0)
  gwait(sa, rows_a, gsem_a)
  pltpu.sync_copy(rows_a, acc_sh.at[dst_v.at[_NCHUNKS - 2]], add=True)
  idx(_NCHUNKS - 1, sb)
  pltpu.async_copy(h_hbm.at[sb], rows_b, gsem_b).wait()
  pltpu.sync_copy(rows_b, acc_sh.at[dst_v.at[_NCHUNKS - 1]], add=True)
  plsc.subcore_barrier()

  rows = pl.ds(r0, _RPT)
  pltpu.sync_copy(acc_sh.at[rows], agg_out.at[cid, rows])

  @pl.when(sid == 0)
  def _copy_rem():
    rem = pl.ds(_NS * _RPT, _REM)
    pltpu.sync_copy(acc_sh.at[rem], agg_out.at[cid, rem])


_sc_agg = pl.kernel(
    _sc_agg_body,
    out_type=jax.ShapeDtypeStruct((_NC, _N, _D), jnp.float32),
    mesh=_MESH,
    scratch_types=[
        pltpu.VMEM((_NCHUNKS, _CHUNK), jnp.int32),
        pltpu.VMEM((_CHUNK,), jnp.int32),
        pltpu.VMEM((_CHUNK,), jnp.int32),
        pltpu.VMEM((_CHUNK, _D), jnp.float32),
        pltpu.VMEM((_CHUNK, _D), jnp.float32),
        pltpu.VMEM_SHARED((_NPAD, _D), jnp.float32),
        pltpu.SemaphoreType.DMA,
        pltpu.SemaphoreType.DMA,
        pltpu.SemaphoreType.DMA,
    ],
    name="sc_sage_agg")


def _sc_counts_body(edges_hbm, z16_hbm, ones_hbm, cnt_out,
                    dst_v, ones_v, cnt_sh, sem):
  cid = lax.axis_index("c")
  sid = lax.axis_index("s")
  wid = sid * _NC + cid

  pltpu.sync_copy(edges_hbm.at[1, wid], dst_v)
  pltpu.sync_copy(ones_hbm, ones_v)

  r0 = pl.multiple_of(sid * _RPT, 8)
  pltpu.sync_copy(z16_hbm, cnt_sh.at[pl.ds(r0, _RPT)])

  @pl.when(sid == 0)
  def _zero_rem():
    pltpu.sync_copy(z16_hbm.at[pl.ds(0, _REM)],
                    cnt_sh.at[pl.ds(_NS * _RPT, _REM)])

  plsc.subcore_barrier()

  def step(c, carry):
    pltpu.async_copy(ones_v, cnt_sh.at[dst_v.at[c]], sem, add=True)
    return carry

  def drain(c, carry):
    pltpu.make_async_copy(ones_v, cnt_sh.at[dst_v.at[0]], sem).wait()
    return carry

  lax.fori_loop(0, _NCHUNKS, step, 0)
  lax.fori_loop(0, _NCHUNKS, drain, 0)
  plsc.subcore_barrier()

  rows = pl.ds(r0, _RPT)
  pltpu.sync_copy(cnt_sh.at[rows], cnt_out.at[cid, rows])

  @pl.when(sid == 0)
  def _copy_rem():
    rem = pl.ds(_NS * _RPT, _REM)
    pltpu.sync_copy(cnt_sh.at[rem], cnt_out.at[cid, rem])


_sc_counts = pl.kernel(
    _sc_counts_body,
    out_type=jax.ShapeDtypeStruct((_NC, _N, _CNTW), jnp.float32),
    mesh=_MESH,
    scratch_types=[
        pltpu.VMEM((_NCHUNKS, _CHUNK), jnp.int32),
        pltpu.VMEM((_CHUNK, _CNTW), jnp.float32),
        pltpu.VMEM_SHARED((_NPAD, _CNTW), jnp.float32),
        pltpu.SemaphoreType.DMA,
    ],
    name="sc_sage_counts")


_BN = 1000
_GRID = _N // _BN


def _dotT(x, w):
  return lax.dot_general(x, w, (((1,), (1,)), ((), ())),
                         preferred_element_type=jnp.float32)


def _tc_pre_body(x_ref, wl_ref, wr_ref, b_ref, h_ref, r_ref):
  x = x_ref[...]
  h_ref[...] = _dotT(x, wl_ref[...])
  r_ref[...] = _dotT(x, wr_ref[...]) + b_ref[...]


_tc_pre = pl.pallas_call(
    _tc_pre_body,
    grid=(_GRID,),
    in_specs=[
        pl.BlockSpec((_BN, _D), lambda i: (i, 0)),
        pl.BlockSpec((_D, _D), lambda i: (0, 0)),
        pl.BlockSpec((_D, _D), lambda i: (0, 0)),
        pl.BlockSpec((1, _D), lambda i: (0, 0)),
    ],
    out_specs=[pl.BlockSpec((_BN, _D), lambda i: (i, 0))] * 2,
    out_shape=[jax.ShapeDtypeStruct((_N, _D), jnp.float32)] * 2,
)


def _tc_mid_body(agg_ref, cnt_ref, r1_ref, wl_ref, wr_ref, b_ref,
                 h_ref, r_ref):
  agg = agg_ref[0] + agg_ref[1]
  cnt = cnt_ref[0, :, 0:1] + cnt_ref[1, :, 0:1]
  x1 = jnp.maximum(agg / jnp.maximum(cnt, 1.0) + r1_ref[...], 0.0)
  h_ref[...] = _dotT(x1, wl_ref[...])
  r_ref[...] = _dotT(x1, wr_ref[...]) + b_ref[...]


_tc_mid = pl.pallas_call(
    _tc_mid_body,
    grid=(_GRID,),
    in_specs=[
        pl.BlockSpec((_NC, _BN, _D), lambda i: (0, i, 0)),
        pl.BlockSpec((_NC, _BN, _CNTW), lambda i: (0, i, 0)),
        pl.BlockSpec((_BN, _D), lambda i: (i, 0)),
        pl.BlockSpec((_D, _D), lambda i: (0, 0)),
        pl.BlockSpec((_D, _D), lambda i: (0, 0)),
        pl.BlockSpec((1, _D), lambda i: (0, 0)),
    ],
    out_specs=[pl.BlockSpec((_BN, _D), lambda i: (i, 0))] * 2,
    out_shape=[jax.ShapeDtypeStruct((_N, _D), jnp.float32)] * 2,
)


def _tc_post_body(agg_ref, cnt_ref, r2_ref, out_ref):
  agg = agg_ref[0] + agg_ref[1]
  cnt = cnt_ref[0, :, 0:1] + cnt_ref[1, :, 0:1]
  out_ref[...] = agg / jnp.maximum(cnt, 1.0) + r2_ref[...]


_tc_post = pl.pallas_call(
    _tc_post_body,
    grid=(_GRID,),
    in_specs=[
        pl.BlockSpec((_NC, _BN, _D), lambda i: (0, i, 0)),
        pl.BlockSpec((_NC, _BN, _CNTW), lambda i: (0, i, 0)),
        pl.BlockSpec((_BN, _D), lambda i: (i, 0)),
    ],
    out_specs=pl.BlockSpec((_BN, _D), lambda i: (i, 0)),
    out_shape=jax.ShapeDtypeStruct((_N, _D), jnp.float32),
)


@jax.jit
def kernel(node_features, edge_indices, W_l1, b_l1, W_r1, W_l2, b_l2, W_r2):
  x = node_features
  e3 = edge_indices.reshape(2, _NW, _EPT)
  npad_e = _EPTP - _EPT
  pad_src = (jnp.arange(npad_e, dtype=jnp.int32)[None, :] * 37
             + jnp.arange(_NW, dtype=jnp.int32)[:, None] * 311) % _N
  src_p = jnp.concatenate([e3[0], pad_src], axis=1)
  trash = _N + (jnp.arange(_NW, dtype=jnp.int32) // _NC)[:, None]
  dst_p = jnp.concatenate(
      [e3[1], jnp.broadcast_to(trash, (_NW, _EPTP - _EPT))], axis=1)
  edges = jnp.stack([src_p, dst_p]).reshape(2, _NW, _NCHUNKS, _CHUNK)
  src_flat = src_p.reshape(_NW * _EPTP)
  z128 = jnp.zeros((_RPT, _D), jnp.float32)
  ones = jnp.ones((_CHUNK, _CNTW), jnp.float32)
  cnt = _sc_counts(edges, z128, ones)
  h1, r1 = _tc_pre(x, W_l1, W_r1, b_l1.reshape(1, _D))
  agg1 = _sc_agg(h1, src_flat, edges, z128)
  h2, r2 = _tc_mid(agg1, cnt, r1, W_l2, W_r2, b_l2.reshape(1, _D))
  agg2 = _sc_agg(h2, src_flat, edges, z128)
  return _tc_post(agg2, cnt, r2)
